# baseline (speedup 1.0000x reference)
import functools

import jax
import jax.numpy as jnp
from jax import lax
from jax.experimental import pallas as pl
from jax.experimental.pallas import tpu as pltpu
from jax.experimental.pallas import tpu_sc as plsc

B = 16384
L = 20
D = 64
NC = 2
NS = 16
LANES = 16
NW = NC * NS
BPW = B // NW
CB = 32
NCHUNK = BPW // CB
GB = 4
NGRP = CB // GB
QPG = GB * L
NRED = QPG // LANES
NIDX = CB * L // 128


def _sc_body(center_hbm, context_hbm, emb_in_hbm, emb_out_hbm, out_hbm,
             cidx_v, xidx_v, in_rows_v, ctx_rows_v, tmp_v, out_v,
             sem_in, sem_ctx):
    wid = lax.axis_index("s") * NC + lax.axis_index("c")
    lane16 = lax.iota(jnp.int32, LANES) * LANES

    def chunk_body(ci, carry):
        base_b = wid * BPW + ci * CB
        pltpu.sync_copy(center_hbm.at[pl.ds(base_b, CB)], cidx_v)
        pltpu.sync_copy(context_hbm.at[pl.ds(base_b * L, CB * L)], xidx_v)
        cp_in = pltpu.async_copy(emb_in_hbm.at[cidx_v], in_rows_v, sem_in)
        cps = [
            pltpu.async_copy(
                emb_out_hbm.at[xidx_v.at[pl.ds(j * 128, 128)]],
                ctx_rows_v.at[pl.ds(j * 128, 128)],
                sem_ctx,
            )
            for j in range(NIDX)
        ]
        cp_in.wait()
        for cp in cps:
            cp.wait()

        def group_body(g4, carry2):
            b0 = g4 * GB
            a = [[in_rows_v[b0 + bb, pl.ds(c * LANES, LANES)]
                  for c in range(D // LANES)] for bb in range(GB)]
            for bb in range(GB):
                for l in range(L):
                    r = (b0 + bb) * L + l
                    q = a[bb][0] * ctx_rows_v[r, pl.ds(0, LANES)]
                    for c in range(1, D // LANES):
                        q = q + a[bb][c] * ctx_rows_v[r, pl.ds(c * LANES, LANES)]
                    tmp_v[pl.ds((bb * L + l) * LANES, LANES)] = q
            for g in range(NRED):
                acc = plsc.load_gather(tmp_v, [lane16 + g * (LANES * LANES)])
                for dd in range(1, LANES):
                    acc = acc + plsc.load_gather(
                        tmp_v, [lane16 + (g * (LANES * LANES) + dd)])
                out_v[pl.ds(g4 * QPG + g * LANES, LANES)] = acc
            return carry2

        lax.fori_loop(0, NGRP, group_body, 0, unroll=False)
        pltpu.sync_copy(out_v, out_hbm.at[pl.ds(base_b * L, CB * L)])
        return carry

    lax.fori_loop(0, NCHUNK, chunk_body, 0, unroll=False)


@functools.partial(jax.jit, static_argnames=())
def _run(center_flat, context_flat, emb_in, emb_out):
    mesh = plsc.VectorSubcoreMesh(
        core_axis_name="c", subcore_axis_name="s",
        num_cores=NC, num_subcores=NS)
    grid_kernel = pl.kernel(
        _sc_body,
        out_type=jax.ShapeDtypeStruct((B * L,), jnp.float32),
        mesh=mesh,
        scratch_types=[
            pltpu.VMEM((CB,), jnp.int32),
            pltpu.VMEM((CB * L,), jnp.int32),
            pltpu.VMEM((CB, D), jnp.float32),
            pltpu.VMEM((CB * L, D), jnp.float32),
            pltpu.VMEM((QPG * LANES,), jnp.float32),
            pltpu.VMEM((CB * L,), jnp.float32),
            pltpu.SemaphoreType.DMA,
            pltpu.SemaphoreType.DMA,
        ],
        compiler_params=pltpu.CompilerParams(
            needs_layout_passes=False, use_tc_tiling_on_sc=False),
    )
    return grid_kernel(center_flat, context_flat, emb_in, emb_out)


def kernel(center, context, emb_in, emb_out):
    center_flat = center.reshape(B)
    context_flat = context.reshape(B * L)
    out_flat = _run(center_flat, context_flat, emb_in, emb_out)
    return out_flat.reshape(B, L)
